# Initial kernel scaffold; baseline (speedup 1.0000x reference)
import jax
import jax.numpy as jnp
from jax.experimental import pallas as pl


def kernel(x, gate_w, gate_b, w1, b1, w2, b2):
    raise NotImplementedError("write your pallas kernel here")


# speedup vs baseline: 1.6239x; 1.6239x over previous
import functools

import jax
import jax.numpy as jnp
from jax import lax
from jax.experimental import pallas as pl
from jax.experimental.pallas import tpu as pltpu

_B, _D, _H, _E = 2048, 768, 2048, 8
_T = 128
_NB = _B // _T
_W = _NB + _E - 1

_INTERPRET = False


def _gate_body(x_ref, gw_ref, gb_ref, top1_ref):
    logits = jnp.dot(x_ref[...], gw_ref[...], preferred_element_type=jnp.float32)
    logits = logits + gb_ref[...]
    scores = jax.nn.softmax(logits, axis=-1)
    top1_ref[...] = jnp.argmax(scores, axis=-1).astype(jnp.int32)[:, None]


def _gating(x, gate_w, gate_b):
    return pl.pallas_call(
        _gate_body,
        out_shape=jax.ShapeDtypeStruct((_B, 1), jnp.int32),
        interpret=_INTERPRET,
    )(x, gate_w, gate_b)


def _build_schedule(counts):
    i32 = jnp.int32
    offsets = jnp.concatenate(
        [jnp.zeros((1,), i32), jnp.cumsum(counts).astype(i32)])
    first_blk = offsets[:_E] // _T
    last_blk = (offsets[1:] - 1) // _T
    nblk = jnp.where(counts > 0, jnp.maximum(last_blk - first_blk + 1, 0), 0)
    cum = jnp.concatenate([jnp.zeros((1,), i32), jnp.cumsum(nblk).astype(i32)])
    i = jnp.arange(_W, dtype=i32)
    eid = jnp.clip(jnp.searchsorted(cum[1:], i, side="right").astype(i32), 0, _E - 1)
    bid = jnp.clip(first_blk[eid] + (i - cum[eid]), 0, _NB - 1)
    start = jnp.clip(offsets[eid] - bid * _T, 0, _T)
    end = jnp.clip(offsets[eid + 1] - bid * _T, 0, _T)
    end = jnp.where(i < cum[-1], end, start)
    first = jnp.concatenate(
        [jnp.ones((1,), jnp.bool_), bid[1:] != bid[:-1]]).astype(i32)
    return jnp.stack([bid, eid, start, end, first])


def _ffn_body(sched_ref, x_ref, w1_ref, b1_ref, w2_ref, b2_ref, out_ref):
    i = pl.program_id(0)
    start = sched_ref[2, i]
    end = sched_ref[3, i]
    first = sched_ref[4, i]
    h = jnp.dot(x_ref[...], w1_ref[0], preferred_element_type=jnp.float32)
    h = jnp.maximum(h + b1_ref[0], 0.0)
    y = jnp.dot(h, w2_ref[0], preferred_element_type=jnp.float32) + b2_ref[0]
    rows = lax.broadcasted_iota(jnp.int32, (_T, 1), 0)
    mask = (rows >= start) & (rows < end)

    @pl.when(first == 1)
    def _():
        out_ref[...] = jnp.where(mask, y, 0.0)

    @pl.when(first == 0)
    def _():
        out_ref[...] = jnp.where(mask, y, out_ref[...])


def _ffn(sched, x_sorted, w1, b1, w2, b2):
    grid_spec = pltpu.PrefetchScalarGridSpec(
        num_scalar_prefetch=1,
        grid=(_W,),
        in_specs=[
            pl.BlockSpec((_T, _D), lambda i, s: (s[0, i], 0)),
            pl.BlockSpec((1, _D, _H), lambda i, s: (s[1, i], 0, 0)),
            pl.BlockSpec((1, 1, _H), lambda i, s: (s[1, i], 0, 0)),
            pl.BlockSpec((1, _H, _D), lambda i, s: (s[1, i], 0, 0)),
            pl.BlockSpec((1, 1, _D), lambda i, s: (s[1, i], 0, 0)),
        ],
        out_specs=pl.BlockSpec((_T, _D), lambda i, s: (s[0, i], 0)),
    )
    return pl.pallas_call(
        _ffn_body,
        grid_spec=grid_spec,
        out_shape=jax.ShapeDtypeStruct((_B, _D), jnp.float32),
        compiler_params=pltpu.CompilerParams(
            dimension_semantics=("arbitrary",)),
        interpret=_INTERPRET,
    )(sched, x_sorted, w1, b1, w2, b2)


def kernel(x, gate_w, gate_b, w1, b1, w2, b2):
    top1 = _gating(x, gate_w, gate_b.reshape(1, _E))[:, 0]
    counts = jnp.bincount(top1, length=_E).astype(jnp.int32)
    sched = _build_schedule(counts)
    sort_idx = jnp.argsort(top1)
    x_sorted = x[sort_idx]
    out_sorted = _ffn(sched, x_sorted, w1,
                      b1.reshape(_E, 1, _H), w2, b2.reshape(_E, 1, _D))
    return jnp.zeros_like(x).at[sort_idx].set(out_sorted)
